# baseline (speedup 1.0000x reference)
import functools

import jax
import jax.numpy as jnp
from jax import lax
from jax.experimental import pallas as pl
from jax.experimental.pallas import tpu as pltpu
from jax.experimental.pallas import tpu_sc as plsc

STATIC_VARS = 26
DIM = 64
BATCH = 4096
B = BATCH * STATIC_VARS
NC, NS = 2, 16
NW = NC * NS
BPW = B // NW
L = 16
SLABR = 416
NSL = BPW // SLABR
NBLK = SLABR // L
K = 2

_MESH = plsc.VectorSubcoreMesh(
    core_axis_name="c", subcore_axis_name="s", num_cores=NC, num_subcores=NS
)


@functools.partial(
    pl.kernel,
    out_type=jax.ShapeDtypeStruct((B * DIM,), jnp.float32),
    mesh=_MESH,
    scratch_types=[
        pltpu.VMEM((STATIC_VARS * DIM,), jnp.float32),
        pltpu.VMEM((BPW,), jnp.int32),
        pltpu.VMEM((K * SLABR * DIM,), jnp.float32),
        pltpu.SemaphoreType.DMA((K,)),
    ],
    compiler_params=pltpu.CompilerParams(
        use_tc_tiling_on_sc=False, needs_layout_passes=False
    ),
)
def _emb_lookup(table_hbm, idx_hbm, out_hbm, table_v, idx_v, bufs, osems):
    wid = lax.axis_index("s") * NC + lax.axis_index("c")
    base = wid * BPW
    pltpu.sync_copy(table_hbm, table_v)
    pltpu.sync_copy(idx_hbm.at[pl.ds(base, BPW)], idx_v)

    lane_iota = lax.iota(jnp.int32, L)
    lane_row = lane_iota * DIM

    def compute_block(slot, s, b):
        idx_vec = idx_v[pl.ds((s * NBLK + b) * L, L)]
        gidx0 = idx_vec * DIM
        sidx0 = (slot * SLABR + b * L) * DIM + lane_row
        for c in range(DIM):
            vals = plsc.load_gather(table_v, [gidx0 + c])
            plsc.store_scatter(bufs, [sidx0 + c], vals)

    def drain_out(slot):
        pltpu.make_async_copy(
            bufs.at[pl.ds(slot * SLABR * DIM, SLABR * DIM)],
            out_hbm.at[pl.ds(base * DIM, SLABR * DIM)],
            osems.at[slot],
        ).wait()

    def body(s, _):
        slot = s % K

        @pl.when(s >= K)
        def _():
            drain_out(slot)

        def blk(b, carry):
            compute_block(slot, s, b)
            return carry

        lax.fori_loop(0, NBLK, blk, 0)
        pltpu.async_copy(
            bufs.at[pl.ds(slot * SLABR * DIM, SLABR * DIM)],
            out_hbm.at[pl.ds((base + s * SLABR) * DIM, SLABR * DIM)],
            osems.at[slot],
        )
        return 0

    lax.fori_loop(0, NSL, body, 0)
    for t in range(K):
        drain_out((NSL - K + t) % K)


def kernel(static_input, table):
    idx = static_input.astype(jnp.int32).reshape(B)
    out = _emb_lookup(table.astype(jnp.float32).reshape(-1), idx)
    return out.reshape(BATCH, STATIC_VARS * DIM)


# speedup vs baseline: 4.6458x; 4.6458x over previous
import functools

import jax
import jax.numpy as jnp
from jax import lax
from jax.experimental import pallas as pl
from jax.experimental.pallas import tpu as pltpu
from jax.experimental.pallas import tpu_sc as plsc

STATIC_VARS = 26
DIM = 64
BATCH = 4096
B = BATCH * STATIC_VARS
NC, NS = 2, 16
NW = NC * NS
BPW = B // NW
L = 16
SLABR = 416
NSL = BPW // SLABR
K = 2
UNROLL = 8

_MESH = plsc.VectorSubcoreMesh(
    core_axis_name="c", subcore_axis_name="s", num_cores=NC, num_subcores=NS
)


@functools.partial(
    pl.kernel,
    out_type=jax.ShapeDtypeStruct((B * DIM,), jnp.float32),
    mesh=_MESH,
    scratch_types=[
        pltpu.VMEM((STATIC_VARS * DIM,), jnp.float32),
        pltpu.VMEM((BPW,), jnp.int32),
        pltpu.VMEM((K * SLABR * DIM,), jnp.float32),
        pltpu.SemaphoreType.DMA((K,)),
    ],
    compiler_params=pltpu.CompilerParams(
        use_tc_tiling_on_sc=False, needs_layout_passes=False
    ),
)
def _emb_lookup(table_hbm, idx_hbm, out_hbm, table_v, idx_v, bufs, osems):
    wid = lax.axis_index("s") * NC + lax.axis_index("c")
    base = wid * BPW
    pltpu.sync_copy(table_hbm, table_v)
    pltpu.sync_copy(idx_hbm.at[pl.ds(base, BPW)], idx_v)

    lane_iota = lax.iota(jnp.int32, L)

    def drain_out(slot):
        pltpu.make_async_copy(
            bufs.at[pl.ds(slot * SLABR * DIM, SLABR * DIM)],
            out_hbm.at[pl.ds(base * DIM, SLABR * DIM)],
            osems.at[slot],
        ).wait()

    def body(s, _):
        slot = s % K

        @pl.when(s >= K)
        def _():
            drain_out(slot)

        srow = s * SLABR
        sbuf = slot * SLABR * DIM

        @plsc.parallel_loop(0, SLABR, unroll=UNROLL)
        def row(r):
            rsplat = plsc.load_gather(
                idx_v, [jnp.full((L,), srow + r, jnp.int32)]
            ).astype(jnp.int32)
            gbase = rsplat * DIM + lane_iota
            obase = sbuf + r * DIM
            for q in range(DIM // L):
                vals = plsc.load_gather(table_v, [gbase + q * L])
                bufs[pl.ds(obase + q * L, L)] = vals

        pltpu.async_copy(
            bufs.at[pl.ds(sbuf, SLABR * DIM)],
            out_hbm.at[pl.ds((base + srow) * DIM, SLABR * DIM)],
            osems.at[slot],
        )
        return 0

    lax.fori_loop(0, NSL, body, 0)
    for t in range(K):
        drain_out((NSL - K + t) % K)


def kernel(static_input, table):
    idx = static_input.astype(jnp.int32).reshape(B)
    out = _emb_lookup(table.astype(jnp.float32).reshape(-1), idx)
    return out.reshape(BATCH, STATIC_VARS * DIM)
